# baseline (speedup 1.0000x reference)
import functools

import jax
import jax.numpy as jnp
from jax import lax
from jax.experimental import pallas as pl
from jax.experimental.pallas import tpu as pltpu
from jax.experimental.pallas import tpu_sc as plsc

VOCAB = 1000000
DIM = 64
BATCH = 16384

_INFO = plsc.get_sparse_core_info()
NC = _INFO.num_cores
NS = _INFO.num_subcores
LANES = _INFO.num_lanes
NW = NC * NS
BPW = BATCH // NW
QE = 128
NQ = BPW // QE

_mesh = plsc.VectorSubcoreMesh(core_axis_name="c", subcore_axis_name="s")


@functools.partial(
    pl.kernel,
    mesh=_mesh,
    compiler_params=pltpu.CompilerParams(needs_layout_passes=False,
                                         use_tc_tiling_on_sc=False),
    out_type=jax.ShapeDtypeStruct((BATCH,), jnp.float32),
    scratch_types=[
        pltpu.VMEM((BPW,), jnp.int32),
        pltpu.VMEM((BPW,), jnp.int32),
        pltpu.VMEM((2, 2, DIM * QE), jnp.int32),
        pltpu.VMEM((2, 2, DIM * QE), jnp.float32),
        pltpu.VMEM((BPW,), jnp.float32),
        pltpu.VMEM((BPW,), jnp.float32),
        pltpu.VMEM((BPW,), jnp.float32),
        pltpu.SemaphoreType.DMA,
        pltpu.SemaphoreType.DMA,
        pltpu.SemaphoreType.DMA,
    ],
)
def _glove_sc(cid_hbm, xid_hbm, wemb_hbm, wctx_hbm, bw_hbm, bc_hbm, out_hbm,
              cid_v, xid_v, idx_v, vals, bw_f, bc_f, out_v,
              sem0, sem1, bsem):
    wid = lax.axis_index("s") * NC + lax.axis_index("c")
    base = wid * BPW

    pltpu.sync_copy(cid_hbm.at[pl.ds(base, BPW)], cid_v)
    pltpu.sync_copy(xid_hbm.at[pl.ds(base, BPW)], xid_v)

    bias_copies = []
    for j in range(NQ):
        sl = pl.ds(j * QE, QE)
        bias_copies.append(pltpu.async_copy(bw_hbm.at[cid_v.at[sl]],
                                            bw_f.at[sl], bsem))
        bias_copies.append(pltpu.async_copy(bc_hbm.at[xid_v.at[sl]],
                                            bc_f.at[sl], bsem))

    iota = lax.iota(jnp.int32, LANES)
    sems = [sem0, sem1]
    tables = [wemb_hbm, wctx_hbm]
    idvs = [cid_v, xid_v]

    def fire(q):
        buf = q % 2
        s = sems[buf]
        for t in range(2):
            def build(g, carry, t=t):
                ids = idvs[t][pl.ds(q * QE + g * LANES, LANES)]

                def per_d(d, carry2, g=g):
                    idx_v.at[buf].at[t][pl.ds(d * QE + g * LANES, LANES)] = (
                        d * VOCAB + ids)
                    return carry2

                lax.fori_loop(0, DIM, per_d, 0)
                return carry

            lax.fori_loop(0, QE // LANES, build, 0)
        for t in range(2):
            for d in range(DIM):
                sl = pl.ds(d * QE, QE)
                pltpu.async_copy(tables[t].at[idx_v.at[buf].at[t].at[sl]],
                                 vals.at[buf].at[t].at[sl], s)

    def drain(q):
        buf = q % 2
        s = sems[buf]
        for t in range(2):
            pltpu.make_async_copy(tables[t].at[pl.ds(0, DIM * QE)],
                                  vals.at[buf].at[t], s).wait()

    fire(0)

    for cp in bias_copies:
        cp.wait()

    for q in range(NQ):
        if q + 1 < NQ:
            fire(q + 1)
        drain(q)
        buf = q % 2
        wq = vals.at[buf].at[0]
        cq = vals.at[buf].at[1]

        def group(g, carry, q=q, wq=wq, cq=cq):
            b0 = q * QE + g * LANES
            acc0 = bw_f[pl.ds(b0, LANES)] + bc_f[pl.ds(b0, LANES)]
            acc1 = jnp.zeros((LANES,), jnp.float32)
            acc2 = jnp.zeros((LANES,), jnp.float32)
            acc3 = jnp.zeros((LANES,), jnp.float32)
            accs = [acc0, acc1, acc2, acc3]
            for d in range(DIM):
                sl = pl.ds(d * QE + g * LANES, LANES)
                accs[d % 4] = accs[d % 4] + wq[sl] * cq[sl]
            out_v[pl.ds(b0, LANES)] = (accs[0] + accs[1]) + (accs[2] + accs[3])
            return carry

        lax.fori_loop(0, QE // LANES, group, 0)

    pltpu.sync_copy(out_v, out_hbm.at[pl.ds(base, BPW)])


def kernel(center_ids, context_ids, W_emb, W_ctx, b_w, b_c):
    cid = center_ids.astype(jnp.int32)
    xid = context_ids.astype(jnp.int32)
    return _glove_sc(cid, xid,
                     W_emb.T.reshape(DIM * VOCAB),
                     W_ctx.T.reshape(DIM * VOCAB),
                     b_w.reshape(VOCAB), b_c.reshape(VOCAB))


# speedup vs baseline: 9.1829x; 9.1829x over previous
import functools

import jax
import jax.numpy as jnp
from jax import lax
from jax.experimental import layout as jax_layout
from jax.experimental import pallas as pl
from jax.experimental.pallas import tpu as pltpu
from jax.experimental.pallas import tpu_sc as plsc

VOCAB = 1000000
DIM = 64
BATCH = 16384

_INFO = plsc.get_sparse_core_info()
NC = _INFO.num_cores
NS = _INFO.num_subcores
LANES = _INFO.num_lanes
NW = NC * NS
BPW = BATCH // NW
CHUNK = 128
NCHUNK = BPW // CHUNK
NGROUP = BPW // LANES

_mesh = plsc.VectorSubcoreMesh(core_axis_name="c", subcore_axis_name="s")


@functools.partial(
    pl.kernel,
    mesh=_mesh,
    compiler_params=pltpu.CompilerParams(needs_layout_passes=False,
                                         use_tc_tiling_on_sc=False),
    out_type=jax.ShapeDtypeStruct((BATCH,), jnp.float32),
    scratch_types=[
        pltpu.VMEM((BPW,), jnp.int32),
        pltpu.VMEM((BPW,), jnp.int32),
        pltpu.VMEM((BPW, DIM), jnp.float32),
        pltpu.VMEM((BPW, DIM), jnp.float32),
        pltpu.VMEM((BPW, 1), jnp.float32),
        pltpu.VMEM((BPW, 1), jnp.float32),
        pltpu.VMEM((BPW,), jnp.float32),
        pltpu.VMEM((BPW,), jnp.float32),
        pltpu.VMEM((BPW,), jnp.float32),
        pltpu.SemaphoreType.DMA,
    ],
)
def _glove_sc(cid_hbm, xid_hbm, wemb_hbm, wctx_hbm, bw_hbm, bc_hbm,
              out_hbm, cid_v, xid_v, wrows, crows, bw_v, bc_v,
              bw_f, bc_f, out_v, sem):
    wid = lax.axis_index("s") * NC + lax.axis_index("c")
    base = wid * BPW

    pltpu.sync_copy(cid_hbm.at[pl.ds(base, BPW)], cid_v)
    pltpu.sync_copy(xid_hbm.at[pl.ds(base, BPW)], xid_v)

    copies = []
    for j in range(NCHUNK):
        sl = pl.ds(j * CHUNK, CHUNK)
        copies.append(pltpu.async_copy(wemb_hbm.at[cid_v.at[sl]],
                                       wrows.at[sl], sem))
        copies.append(pltpu.async_copy(wctx_hbm.at[xid_v.at[sl]],
                                       crows.at[sl], sem))
        copies.append(pltpu.async_copy(bw_hbm.at[cid_v.at[sl]],
                                       bw_f.at[sl], sem))
        copies.append(pltpu.async_copy(bc_hbm.at[xid_v.at[sl]],
                                       bc_f.at[sl], sem))
    for cp in copies:
        cp.wait()

    iota = lax.iota(jnp.int32, LANES)
    def group(g, carry):
        res = bw_f[pl.ds(g * LANES, LANES)] + bc_f[pl.ds(g * LANES, LANES)]
        for u in range(LANES):
            b = g * LANES + u
            wr = wrows.at[b]
            cr = crows.at[b]
            v = wr[pl.ds(0, LANES)] * cr[pl.ds(0, LANES)]
            for k in range(1, DIM // LANES):
                sl = pl.ds(k * LANES, LANES)
                v = v + wr[sl] * cr[sl]
            res = jnp.where(iota == u, res + jnp.sum(v), res)
        out_v[pl.ds(g * LANES, LANES)] = res
        return carry

    lax.fori_loop(0, NGROUP, group, 0)

    pltpu.sync_copy(out_v, out_hbm.at[pl.ds(base, BPW)])


_ROW_MAJOR_2D = jax_layout.Layout(major_to_minor=(0, 1), tiling=((8, 128),))


def kernel(center_ids, context_ids, W_emb, W_ctx, b_w, b_c):
    cid = center_ids.astype(jnp.int32)
    xid = context_ids.astype(jnp.int32)
    wf, cf, bwf, bcf = lax.optimization_barrier(
        (W_emb.reshape(VOCAB * DIM), W_ctx.reshape(VOCAB * DIM),
         b_w.reshape(VOCAB), b_c.reshape(VOCAB)))
    return _glove_sc(cid, xid, wf.reshape(VOCAB, DIM), cf.reshape(VOCAB, DIM),
                     bwf, bcf)
